# baseline (speedup 1.0000x reference)
import functools

import jax
import jax.numpy as jnp
from jax import lax
from jax.experimental import pallas as pl
from jax.experimental.pallas import tpu as pltpu
from jax.experimental.pallas import tpu_sc as plsc

ROWS = 16384
D = 2048
NWORK = 32
RPW = ROWS // NWORK
CHUNK = 16
NCHUNK = RPW // CHUNK
NSL = D // 16
RG = 4


def _vec_sqrt(x):
    y = 0.5 * (1.0 + x)
    for _ in range(12):
        y = 0.5 * (y + x / y)
    return y


def _lane_total(v):
    for m in (1, 2, 4, 8):
        idx = lax.iota(jnp.int32, 16) ^ m
        v = v + jnp.take(v, idx)
    return v


def _compute_chunk(xb, w_v, out_v, c):
    lane = lax.iota(jnp.int32, 16)
    ssqv = jnp.zeros((16,), jnp.float32)
    for g in range(CHUNK // RG):
        zero = jnp.zeros((16,), jnp.float32)
        init = (zero,) * (RG * 8)

        def jbody(j, accs):
            off = pl.multiple_of(j * 16, 16)
            xs = [xb[g * RG + r, pl.ds(off, 16)] for r in range(RG)]
            ws = [w_v[k, pl.ds(off, 16)] for k in range(8)]
            return tuple(accs[r * 8 + k] + xs[r] * ws[k]
                         for r in range(RG) for k in range(8))

        accs = lax.fori_loop(0, NSL, jbody, init, unroll=False)
        for r in range(RG):
            z = jnp.zeros((16,), jnp.float32)
            for k in range(8):
                p = _lane_total(accs[r * 8 + k])
                z = z + p * p
            ssqv = jnp.where(lane == g * RG + r, z, ssqv)
    nrm = _vec_sqrt(ssqv)
    out_v[pl.ds(pl.multiple_of(c * 16, 16), 16)] = nrm


def _sc_body(x_hbm, w_hbm, out_hbm, w_v, xb0, xb1, out_v, sem_w, sem0, sem1, sem_o):
    wid = lax.axis_index("s") * 2 + lax.axis_index("c")
    base = wid * RPW
    pltpu.async_copy(w_hbm, w_v, sem_w).wait()

    def dma(c, buf, sem):
        start = pl.multiple_of(base + c * CHUNK, CHUNK)
        return pltpu.make_async_copy(x_hbm.at[pl.ds(start, CHUNK)], buf, sem)

    dma(0, xb0, sem0).start()

    def outer(i, _):
        c0 = i * 2
        c1 = i * 2 + 1

        @pl.when(c1 < NCHUNK)
        def _():
            dma(c1, xb1, sem1).start()

        dma(c0, xb0, sem0).wait()
        _compute_chunk(xb0, w_v, out_v, c0)

        @pl.when(c0 + 2 < NCHUNK)
        def _():
            dma(c0 + 2, xb0, sem0).start()

        @pl.when(c1 < NCHUNK)
        def _():
            dma(c1, xb1, sem1).wait()
            _compute_chunk(xb1, w_v, out_v, c1)

        return 0

    lax.fori_loop(0, (NCHUNK + 1) // 2, outer, 0, unroll=False)
    pltpu.make_async_copy(out_v, out_hbm.at[pl.ds(base, RPW)], sem_o).start()
    pltpu.make_async_copy(out_v, out_hbm.at[pl.ds(base, RPW)], sem_o).wait()


@functools.partial(jax.jit, static_argnames=())
def kernel_sc_call(x2, w):
    mesh = plsc.VectorSubcoreMesh(core_axis_name="c", subcore_axis_name="s")
    f = pl.kernel(
        _sc_body,
        out_type=jax.ShapeDtypeStruct((ROWS,), jnp.float32),
        mesh=mesh,
        scratch_types=[
            pltpu.VMEM((8, D), jnp.float32),
            pltpu.VMEM((CHUNK, D), jnp.float32),
            pltpu.VMEM((CHUNK, D), jnp.float32),
            pltpu.VMEM((RPW,), jnp.float32),
            pltpu.SemaphoreType.DMA,
            pltpu.SemaphoreType.DMA,
            pltpu.SemaphoreType.DMA,
            pltpu.SemaphoreType.DMA,
        ],
    )
    return f(x2, w)


def kernel(x, routers, expert_idx):
    w = lax.dynamic_index_in_dim(routers, expert_idx, axis=0, keepdims=False)
    out = kernel_sc_call(x.reshape(ROWS, D), w)
    return out.reshape(4, 4096)


# speedup vs baseline: 3.6815x; 3.6815x over previous
import functools

import jax
import jax.numpy as jnp
from jax import lax
from jax.experimental import pallas as pl
from jax.experimental.pallas import tpu as pltpu
from jax.experimental.pallas import tpu_sc as plsc

ROWS = 16384
D = 2048

NWORK = 32
SC_CHUNK = 16
NSL = D // 16
RG = 4
ROWS_SC = 2560
RPW = ROWS_SC // NWORK
NCHUNK = RPW // SC_CHUNK

ROWS_TC = ROWS - ROWS_SC
CH = 512
NCH = ROWS_TC // CH
NBUF = 6


def _vec_sqrt(x):
    y = 0.5 * (1.0 + x)
    for _ in range(14):
        y = 0.5 * (y + x / y)
    return y


def _lane_total(v):
    for m in (1, 2, 4, 8):
        idx = lax.iota(jnp.int32, 16) ^ m
        v = v + jnp.take(v, idx)
    return v


def _sc_compute_chunk(xb, w_v, out_v, c):
    lane = lax.iota(jnp.int32, 16)
    ssqv = jnp.zeros((16,), jnp.float32)
    for g in range(SC_CHUNK // RG):
        zero = jnp.zeros((16,), jnp.float32)
        init = (zero,) * (RG * 8)

        def jbody(j, accs):
            off = pl.multiple_of(j * 16, 16)
            xs = [xb[g * RG + r, pl.ds(off, 16)] for r in range(RG)]
            ws = [w_v[k, pl.ds(off, 16)] for k in range(8)]
            return tuple(accs[r * 8 + k] + xs[r] * ws[k]
                         for r in range(RG) for k in range(8))

        accs = lax.fori_loop(0, NSL, jbody, init, unroll=False)
        for r in range(RG):
            z = jnp.zeros((16,), jnp.float32)
            for k in range(8):
                p = _lane_total(accs[r * 8 + k])
                z = z + p * p
            ssqv = jnp.where(lane == g * RG + r, z, ssqv)
    nrm = _vec_sqrt(ssqv)
    out_v[pl.ds(pl.multiple_of(c * 16, 16), 16)] = nrm


def _sc_body(x_hbm, w_hbm, out_hbm, w_v, xb0, xb1, out_v,
             sem_w, sem0, sem1, sem_o):
    wid = lax.axis_index("s") * 2 + lax.axis_index("c")
    base = wid * RPW
    pltpu.async_copy(w_hbm, w_v, sem_w).wait()

    def dma(c, buf, sem):
        start = pl.multiple_of(base + c * SC_CHUNK, SC_CHUNK)
        return pltpu.make_async_copy(
            x_hbm.at[pl.ds(start, SC_CHUNK)], buf, sem)

    dma(0, xb0, sem0).start()

    def outer(i, _):
        c0 = i * 2
        c1 = i * 2 + 1

        @pl.when(c1 < NCHUNK)
        def _():
            dma(c1, xb1, sem1).start()

        dma(c0, xb0, sem0).wait()
        _sc_compute_chunk(xb0, w_v, out_v, c0)

        @pl.when(c0 + 2 < NCHUNK)
        def _():
            dma(c0 + 2, xb0, sem0).start()

        @pl.when(c1 < NCHUNK)
        def _():
            dma(c1, xb1, sem1).wait()
            _sc_compute_chunk(xb1, w_v, out_v, c1)

        return 0

    lax.fori_loop(0, (NCHUNK + 1) // 2, outer, 0, unroll=False)
    pltpu.make_async_copy(out_v, out_hbm.at[pl.ds(base, RPW)], sem_o).start()
    pltpu.make_async_copy(out_v, out_hbm.at[pl.ds(base, RPW)], sem_o).wait()


def _sc_call(x2, w):
    mesh = plsc.VectorSubcoreMesh(core_axis_name="c", subcore_axis_name="s")
    f = pl.kernel(
        _sc_body,
        out_type=jax.ShapeDtypeStruct((ROWS_SC,), jnp.float32),
        mesh=mesh,
        scratch_types=[
            pltpu.VMEM((8, D), jnp.float32),
            pltpu.VMEM((SC_CHUNK, D), jnp.float32),
            pltpu.VMEM((SC_CHUNK, D), jnp.float32),
            pltpu.VMEM((RPW,), jnp.float32),
            pltpu.SemaphoreType.DMA,
            pltpu.SemaphoreType.DMA,
            pltpu.SemaphoreType.DMA,
            pltpu.SemaphoreType.DMA,
        ],
    )
    return f(x2, w)


def _tc_body(x_hbm, wt_ref, o_ref, xbufs, sems):
    wt = wt_ref[...].astype(jnp.bfloat16)

    def start_dma(c, slot):
        pltpu.make_async_copy(
            x_hbm.at[pl.ds(ROWS_SC + c * CH, CH)], xbufs.at[slot],
            sems.at[slot],
        ).start()

    def wait_dma(c, slot):
        pltpu.make_async_copy(
            x_hbm.at[pl.ds(ROWS_SC + c * CH, CH)], xbufs.at[slot],
            sems.at[slot],
        ).wait()

    for c in range(NBUF - 1):
        start_dma(c, c)

    def step(i, _):
        nxt = i + NBUF - 1

        @pl.when(nxt < NCH)
        def _():
            start_dma(nxt, lax.rem(nxt, NBUF))

        slot = lax.rem(i, NBUF)
        wait_dma(i, slot)
        xb = xbufs[slot].astype(jnp.bfloat16)
        p = jnp.dot(xb, wt, preferred_element_type=jnp.float32)
        o_ref[0, pl.ds(i * CH, CH)] = jnp.sqrt(jnp.sum(p * p, axis=1))
        return 0

    lax.fori_loop(0, NCH, step, 0)


def _tc_call(x2, wt):
    return pl.pallas_call(
        _tc_body,
        in_specs=[
            pl.BlockSpec(memory_space=pl.ANY),
            pl.BlockSpec(memory_space=pltpu.VMEM),
        ],
        out_specs=pl.BlockSpec(memory_space=pltpu.VMEM),
        out_shape=jax.ShapeDtypeStruct((1, ROWS_TC), jnp.float32),
        scratch_shapes=[
            pltpu.VMEM((NBUF, CH, D), jnp.float32),
            pltpu.SemaphoreType.DMA((NBUF,)),
        ],
    )(x2, wt)


def kernel(x, routers, expert_idx):
    w = lax.dynamic_index_in_dim(routers, expert_idx, axis=0,
                                 keepdims=False)
    x2 = x.reshape(ROWS, D)
    sc_out = _sc_call(x2, w)
    tc_out = _tc_call(x2, w.T)
    out = jnp.concatenate([sc_out, tc_out[0]], axis=0)
    return out.reshape(4, 4096)


# speedup vs baseline: 4.7901x; 1.3011x over previous
import jax
import jax.numpy as jnp
from jax import lax
from jax.experimental import pallas as pl
from jax.experimental.pallas import tpu as pltpu

ROWS = 16384
D = 2048
NCORE = 2
RPC = ROWS // NCORE
CH = 512
NCH = RPC // CH
NBUF = 6


def _body(x_hbm, wt_hbm, o_hbm, wt_v, xbufs, obuf, sem_w, sems, sem_o):
    cid = lax.axis_index("x")
    base = cid * RPC
    pltpu.async_copy(wt_hbm, wt_v, sem_w).wait()
    wt = wt_v[...].astype(jnp.bfloat16)

    def start_dma(c, slot):
        pltpu.make_async_copy(
            x_hbm.at[pl.ds(base + c * CH, CH)], xbufs.at[slot],
            sems.at[slot],
        ).start()

    def wait_dma(c, slot):
        pltpu.make_async_copy(
            x_hbm.at[pl.ds(base + c * CH, CH)], xbufs.at[slot],
            sems.at[slot],
        ).wait()

    for c in range(NBUF - 1):
        start_dma(c, c)

    def step(i, _):
        nxt = i + NBUF - 1

        @pl.when(nxt < NCH)
        def _():
            start_dma(nxt, lax.rem(nxt, NBUF))

        slot = lax.rem(i, NBUF)
        wait_dma(i, slot)
        xb = xbufs[slot].astype(jnp.bfloat16)
        p = jnp.dot(xb, wt, preferred_element_type=jnp.float32)
        obuf[0, pl.ds(i * CH, CH)] = jnp.sqrt(jnp.sum(p * p, axis=1))
        return 0

    lax.fori_loop(0, NCH, step, 0)
    cp = pltpu.make_async_copy(obuf, o_hbm.at[:, pl.ds(base, RPC)], sem_o)
    cp.start()
    cp.wait()


def kernel(x, routers, expert_idx):
    w = lax.dynamic_index_in_dim(routers, expert_idx, axis=0,
                                 keepdims=False)
    x2 = x.reshape(ROWS, D)
    mesh = pltpu.create_tensorcore_mesh("x", num_cores=NCORE)
    f = pl.kernel(
        _body,
        out_type=jax.ShapeDtypeStruct((1, ROWS), jnp.float32),
        mesh=mesh,
        scratch_types=[
            pltpu.VMEM((D, 8), jnp.float32),
            pltpu.VMEM((NBUF, CH, D), jnp.float32),
            pltpu.VMEM((1, RPC), jnp.float32),
            pltpu.SemaphoreType.DMA,
            pltpu.SemaphoreType.DMA((NBUF,)),
            pltpu.SemaphoreType.DMA,
        ],
    )
    out = f(x2, w.T)
    return out.reshape(4, 4096)
